# Initial kernel scaffold; baseline (speedup 1.0000x reference)
import jax
import jax.numpy as jnp
from jax.experimental import pallas as pl


def kernel(node_features, edge_index1, edge_weight1, edge_index2, edge_weight2, W_AC1, b_AC1, W_CA1, b_CA1, W_AC2, b_AC2, W_CA2, b_CA2):
    raise NotImplementedError("write your pallas kernel here")


# speedup vs baseline: 1.4740x; 1.4740x over previous
import jax
import jax.numpy as jnp
from jax import lax
from jax.experimental import pallas as pl
from jax.experimental.pallas import tpu as pltpu
from jax.experimental.pallas import tpu_sc as plsc

N = 10000
D = 128
H0 = 128
H1 = 64
E = 320000

NC = 2
NS = 16
L = 16

CHUNK = 128
CPT = 160
EP = CPT * CHUNK * NS
CH = EP // CHUNK
PIECE = 80
NPIECES = N // PIECE
B = 16
NB = CPT // B

_MESH = plsc.VectorSubcoreMesh(
    core_axis_name="c", subcore_axis_name="s", num_cores=NC, num_subcores=NS
)
_SC_PARAMS = pltpu.CompilerParams(needs_layout_passes=False)


def _agg_body(vals_hbm, src_hbm, dst_hbm, wn_hbm,
              out_hbm,
              src_v, dst_v, wn_v, rows_v,
              y_sh, sem):
    c = lax.axis_index("c")
    s = lax.axis_index("s")
    base = c * CH + s * CPT

    z16 = jnp.zeros((L,), jnp.float32)

    @plsc.parallel_loop(0, CHUNK)
    def _zero_rows(i):
        for gz in range(D // L):
            rows_v[i, pl.ds(gz * L, L)] = z16

    for k in range(pl.cdiv(NPIECES, NS)):
        piece = s + k * NS

        @pl.when(piece < NPIECES)
        def _zero_piece():
            pltpu.sync_copy(rows_v.at[pl.ds(0, PIECE)],
                            y_sh.at[pl.ds(piece * PIECE, PIECE)])

    plsc.subcore_barrier()

    def _agg_batch(b, cc):
        b0 = base + b * B
        pltpu.sync_copy(src_hbm.at[pl.ds(b0, B)], src_v)
        pltpu.sync_copy(dst_hbm.at[pl.ds(b0, B)], dst_v)
        pltpu.sync_copy(wn_hbm.at[pl.ds(b0, B)], wn_v)

        for ch in range(B):
            pltpu.async_copy(vals_hbm.at[src_v.at[ch]], rows_v, sem).wait()

            def _scale(e, c3, ch=ch):
                wn_e = plsc.load_gather(
                    wn_v,
                    [jnp.full((L,), ch, jnp.int32),
                     jnp.full((L,), e, jnp.int32)],
                )
                for g in range(D // L):
                    sl = pl.ds(g * L, L)
                    rows_v[e, sl] = rows_v[e, sl] * wn_e
                return c3

            lax.fori_loop(0, CHUNK, _scale, 0)
            pltpu.sync_copy(rows_v, y_sh.at[dst_v.at[ch]], add=True)
        return cc

    lax.fori_loop(0, NB, _agg_batch, 0)

    plsc.subcore_barrier()

    for k in range(pl.cdiv(NPIECES, NS)):
        piece = s + k * NS

        @pl.when(piece < NPIECES)
        def _copy_piece():
            r0 = piece * PIECE
            pltpu.sync_copy(y_sh.at[pl.ds(r0, PIECE)],
                            rows_v.at[pl.ds(0, PIECE)])
            pltpu.sync_copy(rows_v.at[pl.ds(0, PIECE)],
                            out_hbm.at[pl.ds(c * N + r0, PIECE)])


_agg_kernel = pl.kernel(
    _agg_body,
    out_type=jax.ShapeDtypeStruct((NC * N, D), jnp.float32),
    mesh=_MESH,
    compiler_params=_SC_PARAMS,
    scratch_types=[
        pltpu.VMEM((B, CHUNK), jnp.int32),
        pltpu.VMEM((B, CHUNK), jnp.int32),
        pltpu.VMEM((B, CHUNK), jnp.float32),
        pltpu.VMEM((CHUNK, D), jnp.float32),
        pltpu.VMEM_SHARED((N, D), jnp.float32),
        pltpu.SemaphoreType.DMA,
    ],
)


BLK = 400


def _proj_body(y1_ref, y2_ref, wac1_ref, wca1_ref, wac2_ref, wca2_ref,
               b1_ref, b3_ref, pq_ref):
    f32 = jnp.float32
    uu = jnp.concatenate([wac2_ref[0:H0, :], wca2_ref[0:H0, :]], axis=1)
    vv = jnp.concatenate([wac2_ref[H0:, :], wca2_ref[H0:, :]], axis=1)
    ma = jnp.dot(wac1_ref[...], uu, preferred_element_type=f32)
    mb = jnp.dot(wca1_ref[...], vv, preferred_element_type=f32)
    cpq = (jnp.dot(b1_ref[...], uu, preferred_element_type=f32)
           + jnp.dot(b3_ref[...], vv, preferred_element_type=f32))
    pq_ref[...] = (jnp.dot(y1_ref[...], ma, preferred_element_type=f32)
                   + jnp.dot(y2_ref[...], mb, preferred_element_type=f32) + cpq)


_proj_kernel = pl.pallas_call(
    _proj_body,
    grid=(N // BLK,),
    in_specs=[
        pl.BlockSpec((BLK, D), lambda i: (i, 0)),
        pl.BlockSpec((BLK, D), lambda i: (i, 0)),
        pl.BlockSpec((D, H0), lambda i: (0, 0)),
        pl.BlockSpec((D, H0), lambda i: (0, 0)),
        pl.BlockSpec((2 * H0, H1), lambda i: (0, 0)),
        pl.BlockSpec((2 * H0, H1), lambda i: (0, 0)),
        pl.BlockSpec((1, H0), lambda i: (0, 0)),
        pl.BlockSpec((1, H0), lambda i: (0, 0)),
    ],
    out_specs=pl.BlockSpec((BLK, D), lambda i: (i, 0)),
    out_shape=jax.ShapeDtypeStruct((N, D), jnp.float32),
)


def _pad_stack(a1, a2, pad1, pad2, dtype):
    g1 = jnp.concatenate([a1.astype(dtype), pad1])
    g2 = jnp.concatenate([a2.astype(dtype), pad2])
    return jnp.concatenate([g1, g2]).reshape(NC * CH, CHUNK)


def kernel(node_features, edge_index1, edge_weight1, edge_index2, edge_weight2,
           W_AC1, b_AC1, W_CA1, b_CA1, W_AC2, b_AC2, W_CA2, b_CA2):
    pad = EP - E
    pad_idx = (jnp.arange(pad, dtype=jnp.int32) * 37) % N
    zpad = jnp.zeros((pad,), jnp.float32)

    src_all = _pad_stack(edge_index1[0], edge_index2[0], pad_idx, pad_idx, jnp.int32)
    dst_all = _pad_stack(edge_index1[1], edge_index2[1], pad_idx, pad_idx, jnp.int32)

    def _wn(src, dst, w):
        deg = jnp.zeros((N,), jnp.float32).at[dst].add(w)
        norm = lax.rsqrt(jnp.clip(deg, 1e-12, None))
        return w * norm[src] * norm[dst]

    wn1 = _wn(edge_index1[0], edge_index1[1], edge_weight1)
    wn2 = _wn(edge_index2[0], edge_index2[1], edge_weight2)
    wn_all = _pad_stack(wn1, wn2, zpad, zpad, jnp.float32)

    y = _agg_kernel(node_features, src_all, dst_all, wn_all)

    pq = _proj_kernel(y[:N], y[N:], W_AC1, W_CA1, W_AC2, W_CA2,
                      b_AC1.reshape(1, H0), b_CA1.reshape(1, H0))

    out2 = _agg_kernel(pq, src_all, dst_all, wn_all)

    return jnp.concatenate([out2[:N, :H1] + b_AC2,
                            out2[N:, H1:] + b_CA2], axis=1)


# speedup vs baseline: 1.5020x; 1.0190x over previous
import jax
import jax.numpy as jnp
from jax import lax
from jax.experimental import pallas as pl
from jax.experimental.pallas import tpu as pltpu
from jax.experimental.pallas import tpu_sc as plsc

N = 10000
D = 128
H0 = 128
H1 = 64
E = 320000

NC = 2
NS = 16
L = 16

CHUNK = 128
CPT = 160
EP = CPT * CHUNK * NS
CH = EP // CHUNK
PIECE = 80
NPIECES = N // PIECE
B = 16
NB = CPT // B

_MESH = plsc.VectorSubcoreMesh(
    core_axis_name="c", subcore_axis_name="s", num_cores=NC, num_subcores=NS
)
_SC_PARAMS = pltpu.CompilerParams(needs_layout_passes=False)


def _agg_body(vals_hbm, src_hbm, dst_hbm, wn_hbm,
              out_hbm,
              src_v, dst_v, wn_v, rows_v,
              y_sh, sem):
    c = lax.axis_index("c")
    s = lax.axis_index("s")
    base = c * CH + s * CPT

    z16 = jnp.zeros((L,), jnp.float32)

    @plsc.parallel_loop(0, CHUNK)
    def _zero_rows(i):
        for gz in range(D // L):
            rows_v[i, pl.ds(gz * L, L)] = z16

    for k in range(pl.cdiv(NPIECES, NS)):
        piece = s + k * NS

        @pl.when(piece < NPIECES)
        def _zero_piece():
            pltpu.sync_copy(rows_v.at[pl.ds(0, PIECE)],
                            y_sh.at[pl.ds(piece * PIECE, PIECE)])

    plsc.subcore_barrier()

    def _agg_batch(b, cc):
        b0 = base + b * B
        pltpu.sync_copy(src_hbm.at[pl.ds(b0, B)], src_v)
        pltpu.sync_copy(dst_hbm.at[pl.ds(b0, B)], dst_v)
        pltpu.sync_copy(wn_hbm.at[pl.ds(b0, B)], wn_v)

        for ch in range(B):
            pltpu.async_copy(vals_hbm.at[src_v.at[ch]], rows_v, sem).wait()

            @plsc.parallel_loop(0, CHUNK)
            def _scale(e, ch=ch):
                wn_e = plsc.load_gather(
                    wn_v,
                    [jnp.full((L,), ch, jnp.int32),
                     jnp.full((L,), e, jnp.int32)],
                )
                for g in range(D // L):
                    sl = pl.ds(g * L, L)
                    rows_v[e, sl] = rows_v[e, sl] * wn_e
            pltpu.sync_copy(rows_v, y_sh.at[dst_v.at[ch]], add=True)
        return cc

    lax.fori_loop(0, NB, _agg_batch, 0)

    plsc.subcore_barrier()

    for k in range(pl.cdiv(NPIECES, NS)):
        piece = s + k * NS

        @pl.when(piece < NPIECES)
        def _copy_piece():
            r0 = piece * PIECE
            pltpu.sync_copy(y_sh.at[pl.ds(r0, PIECE)],
                            rows_v.at[pl.ds(0, PIECE)])
            pltpu.sync_copy(rows_v.at[pl.ds(0, PIECE)],
                            out_hbm.at[pl.ds(c * N + r0, PIECE)])


_agg_kernel = pl.kernel(
    _agg_body,
    out_type=jax.ShapeDtypeStruct((NC * N, D), jnp.float32),
    mesh=_MESH,
    compiler_params=_SC_PARAMS,
    scratch_types=[
        pltpu.VMEM((B, CHUNK), jnp.int32),
        pltpu.VMEM((B, CHUNK), jnp.int32),
        pltpu.VMEM((B, CHUNK), jnp.float32),
        pltpu.VMEM((CHUNK, D), jnp.float32),
        pltpu.VMEM_SHARED((N, D), jnp.float32),
        pltpu.SemaphoreType.DMA,
    ],
)


BLK = 400


def _proj_body(y1_ref, y2_ref, wac1_ref, wca1_ref, wac2_ref, wca2_ref,
               b1_ref, b3_ref, pq_ref):
    f32 = jnp.float32
    uu = jnp.concatenate([wac2_ref[0:H0, :], wca2_ref[0:H0, :]], axis=1)
    vv = jnp.concatenate([wac2_ref[H0:, :], wca2_ref[H0:, :]], axis=1)
    ma = jnp.dot(wac1_ref[...], uu, preferred_element_type=f32)
    mb = jnp.dot(wca1_ref[...], vv, preferred_element_type=f32)
    cpq = (jnp.dot(b1_ref[...], uu, preferred_element_type=f32)
           + jnp.dot(b3_ref[...], vv, preferred_element_type=f32))
    pq_ref[...] = (jnp.dot(y1_ref[...], ma, preferred_element_type=f32)
                   + jnp.dot(y2_ref[...], mb, preferred_element_type=f32) + cpq)


_proj_kernel = pl.pallas_call(
    _proj_body,
    grid=(N // BLK,),
    in_specs=[
        pl.BlockSpec((BLK, D), lambda i: (i, 0)),
        pl.BlockSpec((BLK, D), lambda i: (i, 0)),
        pl.BlockSpec((D, H0), lambda i: (0, 0)),
        pl.BlockSpec((D, H0), lambda i: (0, 0)),
        pl.BlockSpec((2 * H0, H1), lambda i: (0, 0)),
        pl.BlockSpec((2 * H0, H1), lambda i: (0, 0)),
        pl.BlockSpec((1, H0), lambda i: (0, 0)),
        pl.BlockSpec((1, H0), lambda i: (0, 0)),
    ],
    out_specs=pl.BlockSpec((BLK, D), lambda i: (i, 0)),
    out_shape=jax.ShapeDtypeStruct((N, D), jnp.float32),
)


def _pad_stack(a1, a2, pad1, pad2, dtype):
    g1 = jnp.concatenate([a1.astype(dtype), pad1])
    g2 = jnp.concatenate([a2.astype(dtype), pad2])
    return jnp.concatenate([g1, g2]).reshape(NC * CH, CHUNK)


def kernel(node_features, edge_index1, edge_weight1, edge_index2, edge_weight2,
           W_AC1, b_AC1, W_CA1, b_CA1, W_AC2, b_AC2, W_CA2, b_CA2):
    pad = EP - E
    pad_idx = (jnp.arange(pad, dtype=jnp.int32) * 37) % N
    zpad = jnp.zeros((pad,), jnp.float32)

    src_all = _pad_stack(edge_index1[0], edge_index2[0], pad_idx, pad_idx, jnp.int32)
    dst_all = _pad_stack(edge_index1[1], edge_index2[1], pad_idx, pad_idx, jnp.int32)

    def _wn(src, dst, w):
        deg = jnp.zeros((N,), jnp.float32).at[dst].add(w)
        norm = lax.rsqrt(jnp.clip(deg, 1e-12, None))
        return w * norm[src] * norm[dst]

    wn1 = _wn(edge_index1[0], edge_index1[1], edge_weight1)
    wn2 = _wn(edge_index2[0], edge_index2[1], edge_weight2)
    wn_all = _pad_stack(wn1, wn2, zpad, zpad, jnp.float32)

    y = _agg_kernel(node_features, src_all, dst_all, wn_all)

    pq = _proj_kernel(y[:N], y[N:], W_AC1, W_CA1, W_AC2, W_CA2,
                      b_AC1.reshape(1, H0), b_CA1.reshape(1, H0))

    out2 = _agg_kernel(pq, src_all, dst_all, wn_all)

    return jnp.concatenate([out2[:N, :H1] + b_AC2,
                            out2[N:, H1:] + b_CA2], axis=1)


# speedup vs baseline: 1.5517x; 1.0331x over previous
import jax
import jax.numpy as jnp
from jax import lax
from jax.experimental import pallas as pl
from jax.experimental.pallas import tpu as pltpu
from jax.experimental.pallas import tpu_sc as plsc

N = 10000
D = 128
H0 = 128
H1 = 64
E = 320000

NC = 2
NS = 16
L = 16

CHUNK = 128
CPT = 160
EP = CPT * CHUNK * NS
CH = EP // CHUNK
PIECE = 80
NPIECES = N // PIECE
B = 16
NB = CPT // B

_MESH = plsc.VectorSubcoreMesh(
    core_axis_name="c", subcore_axis_name="s", num_cores=NC, num_subcores=NS
)
_SC_PARAMS = pltpu.CompilerParams(needs_layout_passes=False)


def _agg_body(vals_hbm, src_hbm, dst_hbm, wn_hbm,
              out_hbm,
              src_v, dst_v, wn_v, rows_v, rows2_v,
              y_sh, gsem0, gsem1, ssem0, ssem1):
    c = lax.axis_index("c")
    s = lax.axis_index("s")
    base = c * CH + s * CPT

    z16 = jnp.zeros((L,), jnp.float32)

    @plsc.parallel_loop(0, CHUNK)
    def _zero_rows(i):
        for gz in range(D // L):
            rows_v[i, pl.ds(gz * L, L)] = z16

    for k in range(pl.cdiv(NPIECES, NS)):
        piece = s + k * NS

        @pl.when(piece < NPIECES)
        def _zero_piece():
            pltpu.sync_copy(rows_v.at[pl.ds(0, PIECE)],
                            y_sh.at[pl.ds(piece * PIECE, PIECE)])

    plsc.subcore_barrier()

    bufs = (rows_v, rows2_v)
    gsems = (gsem0, gsem1)
    ssems = (ssem0, ssem1)

    def _agg_batch(b, cc):
        b0 = base + b * B
        pltpu.sync_copy(src_hbm.at[pl.ds(b0, B)], src_v)
        pltpu.sync_copy(dst_hbm.at[pl.ds(b0, B)], dst_v)
        pltpu.sync_copy(wn_hbm.at[pl.ds(b0, B)], wn_v)

        gathers = [None] * B
        scatters = [None] * B
        gathers[0] = pltpu.async_copy(
            vals_hbm.at[src_v.at[0]], bufs[0], gsems[0])
        for ch in range(B):
            k = ch % 2
            if ch + 1 < B:
                if ch >= 1:
                    scatters[ch - 1].wait()
                gathers[ch + 1] = pltpu.async_copy(
                    vals_hbm.at[src_v.at[ch + 1]],
                    bufs[(ch + 1) % 2], gsems[(ch + 1) % 2])
            gathers[ch].wait()

            @plsc.parallel_loop(0, CHUNK)
            def _scale(e, ch=ch, k=k):
                wn_e = plsc.load_gather(
                    wn_v,
                    [jnp.full((L,), ch, jnp.int32),
                     jnp.full((L,), e, jnp.int32)],
                )
                for g in range(D // L):
                    sl = pl.ds(g * L, L)
                    bufs[k][e, sl] = bufs[k][e, sl] * wn_e

            scatters[ch] = pltpu.async_copy(
                bufs[k], y_sh.at[dst_v.at[ch]], ssems[k], add=True)
        scatters[B - 2].wait()
        scatters[B - 1].wait()
        return cc

    lax.fori_loop(0, NB, _agg_batch, 0)

    plsc.subcore_barrier()

    for k in range(pl.cdiv(NPIECES, NS)):
        piece = s + k * NS

        @pl.when(piece < NPIECES)
        def _copy_piece():
            r0 = piece * PIECE
            pltpu.sync_copy(y_sh.at[pl.ds(r0, PIECE)],
                            rows_v.at[pl.ds(0, PIECE)])
            pltpu.sync_copy(rows_v.at[pl.ds(0, PIECE)],
                            out_hbm.at[pl.ds(c * N + r0, PIECE)])


_agg_kernel = pl.kernel(
    _agg_body,
    out_type=jax.ShapeDtypeStruct((NC * N, D), jnp.float32),
    mesh=_MESH,
    compiler_params=_SC_PARAMS,
    scratch_types=[
        pltpu.VMEM((B, CHUNK), jnp.int32),
        pltpu.VMEM((B, CHUNK), jnp.int32),
        pltpu.VMEM((B, CHUNK), jnp.float32),
        pltpu.VMEM((CHUNK, D), jnp.float32),
        pltpu.VMEM((CHUNK, D), jnp.float32),
        pltpu.VMEM_SHARED((N, D), jnp.float32),
        pltpu.SemaphoreType.DMA,
        pltpu.SemaphoreType.DMA,
        pltpu.SemaphoreType.DMA,
        pltpu.SemaphoreType.DMA,
    ],
)


BLK = 400


def _proj_body(y1_ref, y2_ref, wac1_ref, wca1_ref, wac2_ref, wca2_ref,
               b1_ref, b3_ref, pq_ref):
    f32 = jnp.float32
    uu = jnp.concatenate([wac2_ref[0:H0, :], wca2_ref[0:H0, :]], axis=1)
    vv = jnp.concatenate([wac2_ref[H0:, :], wca2_ref[H0:, :]], axis=1)
    ma = jnp.dot(wac1_ref[...], uu, preferred_element_type=f32)
    mb = jnp.dot(wca1_ref[...], vv, preferred_element_type=f32)
    cpq = (jnp.dot(b1_ref[...], uu, preferred_element_type=f32)
           + jnp.dot(b3_ref[...], vv, preferred_element_type=f32))
    pq_ref[...] = (jnp.dot(y1_ref[...], ma, preferred_element_type=f32)
                   + jnp.dot(y2_ref[...], mb, preferred_element_type=f32) + cpq)


_proj_kernel = pl.pallas_call(
    _proj_body,
    grid=(N // BLK,),
    in_specs=[
        pl.BlockSpec((BLK, D), lambda i: (i, 0)),
        pl.BlockSpec((BLK, D), lambda i: (i, 0)),
        pl.BlockSpec((D, H0), lambda i: (0, 0)),
        pl.BlockSpec((D, H0), lambda i: (0, 0)),
        pl.BlockSpec((2 * H0, H1), lambda i: (0, 0)),
        pl.BlockSpec((2 * H0, H1), lambda i: (0, 0)),
        pl.BlockSpec((1, H0), lambda i: (0, 0)),
        pl.BlockSpec((1, H0), lambda i: (0, 0)),
    ],
    out_specs=pl.BlockSpec((BLK, D), lambda i: (i, 0)),
    out_shape=jax.ShapeDtypeStruct((N, D), jnp.float32),
)


def _pad_stack(a1, a2, pad1, pad2, dtype):
    g1 = jnp.concatenate([a1.astype(dtype), pad1])
    g2 = jnp.concatenate([a2.astype(dtype), pad2])
    return jnp.concatenate([g1, g2]).reshape(NC * CH, CHUNK)


def kernel(node_features, edge_index1, edge_weight1, edge_index2, edge_weight2,
           W_AC1, b_AC1, W_CA1, b_CA1, W_AC2, b_AC2, W_CA2, b_CA2):
    pad = EP - E
    pad_idx = (jnp.arange(pad, dtype=jnp.int32) * 37) % N
    zpad = jnp.zeros((pad,), jnp.float32)

    src_all = _pad_stack(edge_index1[0], edge_index2[0], pad_idx, pad_idx, jnp.int32)
    dst_all = _pad_stack(edge_index1[1], edge_index2[1], pad_idx, pad_idx, jnp.int32)

    def _wn(src, dst, w):
        deg = jnp.zeros((N,), jnp.float32).at[dst].add(w)
        norm = lax.rsqrt(jnp.clip(deg, 1e-12, None))
        return w * norm[src] * norm[dst]

    wn1 = _wn(edge_index1[0], edge_index1[1], edge_weight1)
    wn2 = _wn(edge_index2[0], edge_index2[1], edge_weight2)
    wn_all = _pad_stack(wn1, wn2, zpad, zpad, jnp.float32)

    y = _agg_kernel(node_features, src_all, dst_all, wn_all)

    pq = _proj_kernel(y[:N], y[N:], W_AC1, W_CA1, W_AC2, W_CA2,
                      b_AC1.reshape(1, H0), b_CA1.reshape(1, H0))

    out2 = _agg_kernel(pq, src_all, dst_all, wn_all)

    return jnp.concatenate([out2[:N, :H1] + b_AC2,
                            out2[N:, H1:] + b_CA2], axis=1)
